# baseline (speedup 1.0000x reference)
import functools
import jax
import jax.numpy as jnp
from jax import lax
from jax.experimental import pallas as pl
from jax.experimental.pallas import tpu as pltpu
from jax.experimental.pallas import tpu_sc as plsc

N = 10000
E = 320000
D = 128
H = 8
HD = 16

NC = 2
NS = 16
NW = NC * NS
EPW = E // NW
BE = 40
NBPW = EPW // BE
ZC = 64
NZFULL = N // ZC
ZREM = N - NZFULL * ZC
SCALE = 1.0 / (4.0 * 32.0)


def _edge_mlp_body(ef, chi, w1, b1, w2, b2, w3, b3, w4, b4, out):
    h1 = jnp.dot(ef[...], w1[...], preferred_element_type=jnp.float32) + b1[...]
    h1 = h1 * jax.nn.sigmoid(h1)
    r = jnp.dot(h1, w2[...], preferred_element_type=jnp.float32) + b2[...]
    h2 = jnp.dot(chi[...], w3[...], preferred_element_type=jnp.float32) + b3[...]
    h2 = h2 * jax.nn.sigmoid(h2)
    out[...] = r + jnp.dot(h2, w4[...], preferred_element_type=jnp.float32) + b4[...]


def _edge_mlp(ef, chi, w1, b1, w2, b2, w3, b3, w4, b4):
    BEW = 1600
    grid = E // BEW
    full = lambda shape: pl.BlockSpec(shape, lambda i: (0, 0))
    return pl.pallas_call(
        _edge_mlp_body,
        grid=(grid,),
        in_specs=[
            pl.BlockSpec((BEW, 16), lambda i: (i, 0)),
            pl.BlockSpec((BEW, 16), lambda i: (i, 0)),
            full((16, 64)), full((1, 64)),
            full((64, D)), full((1, D)),
            full((16, 64)), full((1, 64)),
            full((64, D)), full((1, D)),
        ],
        out_specs=pl.BlockSpec((BEW, D), lambda i: (i, 0)),
        out_shape=jax.ShapeDtypeStruct((E, D), jnp.float32),
    )(ef, chi, w1, b1, w2, b2, w3, b3, w4, b4)


def _qkv_body(nf, wq, wk, wv, qo, ko, vo):
    x = nf[...]
    qo[...] = jnp.dot(x, wq[...], preferred_element_type=jnp.float32)
    ko[...] = jnp.dot(x, wk[...], preferred_element_type=jnp.float32)
    vo[...] = jnp.dot(x, wv[...], preferred_element_type=jnp.float32)


def _qkv(nf, wq, wk, wv):
    BN = 1000
    grid = N // BN
    full = lambda: pl.BlockSpec((D, D), lambda i: (0, 0))
    s = jax.ShapeDtypeStruct((N, D), jnp.float32)
    return pl.pallas_call(
        _qkv_body,
        grid=(grid,),
        in_specs=[pl.BlockSpec((BN, D), lambda i: (i, 0)), full(), full(), full()],
        out_specs=[pl.BlockSpec((BN, D), lambda i: (i, 0))] * 3,
        out_shape=[s, s, s],
    )(nf, wq, wk, wv)


def _sc_body(wp, qt, kt, vt, snd, rcv, cut, out,
             sidx0, ridx0, cutv0, wv0, qv0, kv0, vv0,
             sidx1, ridx1, cutv1, wv1, qv1, kv1, vv1,
             c_v, acc, semi0, semg0, semi1, semg1):
    c = lax.axis_index("c")
    s = lax.axis_index("s")
    wid = c * NS + s
    base0 = wid * EPW

    sets = ((sidx0, ridx0, cutv0, wv0, qv0, kv0, vv0, semi0, semg0),
            (sidx1, ridx1, cutv1, wv1, qv1, kv1, vv1, semi1, semg1))

    def zrow(r, carry):
        for j in range(D // 16):
            c_v[r, pl.ds(j * 16, 16)] = jnp.zeros((16,), jnp.float32)
        return carry
    lax.fori_loop(0, ZC, zrow, 0)

    def zchunk(j, carry):
        i = s + j * NS
        @pl.when(i < NZFULL)
        def _():
            pltpu.sync_copy(c_v, acc.at[pl.ds(i * ZC, ZC)])
        return carry
    lax.fori_loop(0, (NZFULL + NS - 1) // NS, zchunk, 0)

    @pl.when(s == 0)
    def _():
        pltpu.sync_copy(c_v.at[pl.ds(0, ZREM)], acc.at[pl.ds(NZFULL * ZC, ZREM)])
    plsc.subcore_barrier()

    def issue_idx(i, st):
        sidx, ridx, cutv = st[0], st[1], st[2]
        semi = st[7]
        b = base0 + i * BE
        pltpu.async_copy(snd.at[pl.ds(b, BE)], sidx, semi)
        pltpu.async_copy(rcv.at[pl.ds(b, BE)], ridx, semi)
        pltpu.async_copy(cut.at[pl.ds(b, BE)], cutv.at[pl.ds(0, BE)], semi)

    def wait_idx(st):
        sidx, ridx, cutv = st[0], st[1], st[2]
        semi = st[7]
        pltpu.make_async_copy(snd.at[pl.ds(0, BE)], sidx, semi).wait()
        pltpu.make_async_copy(rcv.at[pl.ds(0, BE)], ridx, semi).wait()
        pltpu.make_async_copy(cut.at[pl.ds(0, BE)], cutv.at[pl.ds(0, BE)], semi).wait()

    def issue_gathers(i, st):
        sidx, ridx, wv, qv, kv, vv = st[0], st[1], st[3], st[4], st[5], st[6]
        semg = st[8]
        b = base0 + i * BE
        pltpu.async_copy(wp.at[pl.ds(b, BE)], wv, semg)
        pltpu.async_copy(qt.at[ridx], qv, semg)
        pltpu.async_copy(kt.at[sidx], kv, semg)
        pltpu.async_copy(vt.at[sidx], vv, semg)

    def wait_gathers(st):
        semg = st[8]
        for dst in (st[3], st[4], st[5], st[6]):
            pltpu.make_async_copy(wp.at[pl.ds(0, BE)], dst, semg).wait()

    lanes = lax.iota(jnp.int32, 16)
    perms = [lanes ^ jnp.int32(1 << jj) for jj in (3, 2, 1, 0)]
    dnums = lax.GatherDimensionNumbers(
        offset_dims=(), collapsed_slice_dims=(0,), start_index_map=(0,))

    def lane_perm(x, pm):
        return lax.gather(x, pm[:, None], dnums, slice_sizes=(1,),
                          mode=lax.GatherScatterMode.PROMISE_IN_BOUNDS)

    def compute_block(st):
        ridx, cutv, wv, qv, kv, vv = st[1], st[2], st[3], st[4], st[5], st[6]

        def edge(e, carry2):
            wbase = e & ~jnp.int32(15)
            lane = e & jnp.int32(15)
            cvec = cutv[pl.ds(wbase, 16)]
            scale = lane_perm(cvec, jnp.full((16,), 0, jnp.int32) + lane) * SCALE
            for h in range(H):
                dsl = pl.ds(h * HD, HD)
                p = qv[e, dsl] * wv[e, dsl] * kv[e, dsl]
                for pm in perms:
                    p = p + lane_perm(p, pm)
                c_v[e, dsl] = (p * scale) * vv[e, dsl]
            return carry2
        lax.fori_loop(0, BE, edge, 0)
        pltpu.sync_copy(c_v.at[pl.ds(0, BE)], acc.at[ridx], add=True)

    issue_idx(0, sets[0])
    wait_idx(sets[0])
    issue_gathers(0, sets[0])
    issue_idx(1, sets[1])

    def pair(t, carry):
        for par in (0, 1):
            i = 2 * t + par
            cur, nxt = sets[par], sets[1 - par]

            @pl.when(i + 1 < NBPW)
            def _():
                wait_idx(nxt)
                issue_gathers(i + 1, nxt)
            wait_gathers(cur)
            compute_block(cur)

            @pl.when(i + 2 < NBPW)
            def _():
                issue_idx(i + 2, cur)
        return carry
    lax.fori_loop(0, NBPW // 2, pair, 0)

    plsc.subcore_barrier()

    def ochunk(j, carry):
        i = s + j * NS
        @pl.when(i < NZFULL)
        def _():
            r0 = i * ZC
            pltpu.sync_copy(acc.at[pl.ds(r0, ZC)], c_v)
            pltpu.sync_copy(c_v, out.at[c, pl.ds(r0, ZC)])
        return carry
    lax.fori_loop(0, (NZFULL + NS - 1) // NS, ochunk, 0)

    @pl.when(s == 0)
    def _():
        r0 = NZFULL * ZC
        pltpu.sync_copy(acc.at[pl.ds(r0, ZREM)], c_v.at[pl.ds(0, ZREM)])
        pltpu.sync_copy(c_v.at[pl.ds(0, ZREM)], out.at[c, pl.ds(r0, ZREM)])


def _sc_scatter(wp, qt, kt, vt, snd, rcv, cut):
    mesh = plsc.VectorSubcoreMesh(core_axis_name="c", subcore_axis_name="s")
    buf = lambda: [
        pltpu.VMEM((BE,), jnp.int32),
        pltpu.VMEM((BE,), jnp.int32),
        pltpu.VMEM((BE + 16,), jnp.float32),
        pltpu.VMEM((BE, D), jnp.float32),
        pltpu.VMEM((BE, D), jnp.float32),
        pltpu.VMEM((BE, D), jnp.float32),
        pltpu.VMEM((BE, D), jnp.float32),
    ]
    f = functools.partial(
        pl.kernel,
        out_type=jax.ShapeDtypeStruct((NC, N, D), jnp.float32),
        mesh=mesh,
        scratch_types=buf() + buf() + [
            pltpu.VMEM((ZC, D), jnp.float32),
            pltpu.VMEM_SHARED((N, D), jnp.float32),
            pltpu.SemaphoreType.DMA,
            pltpu.SemaphoreType.DMA,
            pltpu.SemaphoreType.DMA,
            pltpu.SemaphoreType.DMA,
        ],
    )(_sc_body)
    return f(wp, qt, kt, vt, snd, rcv, cut)


def _sum_body(p, o):
    o[...] = p[0] + p[1]


def _sum_partials(parts):
    BN = 1000
    return pl.pallas_call(
        _sum_body,
        grid=(N // BN,),
        in_specs=[pl.BlockSpec((NC, BN, D), lambda i: (0, i, 0))],
        out_specs=pl.BlockSpec((BN, D), lambda i: (i, 0)),
        out_shape=jax.ShapeDtypeStruct((N, D), jnp.float32),
    )(parts)


def kernel(node_feats, edge_feats, chi_scalar, cutoffs, senders, receivers,
           W_rad1, b_rad1, W_rad2, b_rad2,
           W_sph1, b_sph1, W_sph2, b_sph2,
           Wq, Wk, Wv):
    wp = _edge_mlp(edge_feats, chi_scalar,
                   W_rad1, b_rad1.reshape(1, 64), W_rad2, b_rad2.reshape(1, D),
                   W_sph1, b_sph1.reshape(1, 64), W_sph2, b_sph2.reshape(1, D))
    qt, kt, vt = _qkv(node_feats, Wq, Wk, Wv)
    parts = _sc_scatter(wp, qt, kt, vt,
                        senders.astype(jnp.int32), receivers.astype(jnp.int32),
                        cutoffs.astype(jnp.float32))
    return _sum_partials(parts)


# speedup vs baseline: 1.7779x; 1.7779x over previous
import functools
import jax
import jax.numpy as jnp
from jax import lax
from jax.experimental import pallas as pl
from jax.experimental.pallas import tpu as pltpu
from jax.experimental.pallas import tpu_sc as plsc

N = 10000
E = 320000
D = 128
H = 8
HD = 16

NC = 2
NS = 16
NW = NC * NS
EPW = E // NW
BE = 40
NBPW = EPW // BE
ZC = 64
NZFULL = N // ZC
ZREM = N - NZFULL * ZC
SCALE = 1.0 / (4.0 * 32.0)


def _edge_mlp_body(ef, chi, w1, b1, w2, b2, w3, b3, w4, b4, out):
    h1 = jnp.dot(ef[...], w1[...], preferred_element_type=jnp.float32) + b1[...]
    h1 = h1 * jax.nn.sigmoid(h1)
    r = jnp.dot(h1, w2[...], preferred_element_type=jnp.float32) + b2[...]
    h2 = jnp.dot(chi[...], w3[...], preferred_element_type=jnp.float32) + b3[...]
    h2 = h2 * jax.nn.sigmoid(h2)
    out[...] = r + jnp.dot(h2, w4[...], preferred_element_type=jnp.float32) + b4[...]


def _edge_mlp(ef, chi, w1, b1, w2, b2, w3, b3, w4, b4):
    BEW = 1600
    grid = E // BEW
    full = lambda shape: pl.BlockSpec(shape, lambda i: (0, 0))
    return pl.pallas_call(
        _edge_mlp_body,
        grid=(grid,),
        in_specs=[
            pl.BlockSpec((BEW, 16), lambda i: (i, 0)),
            pl.BlockSpec((BEW, 16), lambda i: (i, 0)),
            full((16, 64)), full((1, 64)),
            full((64, D)), full((1, D)),
            full((16, 64)), full((1, 64)),
            full((64, D)), full((1, D)),
        ],
        out_specs=pl.BlockSpec((BEW, D), lambda i: (i, 0)),
        out_shape=jax.ShapeDtypeStruct((E, D), jnp.float32),
    )(ef, chi, w1, b1, w2, b2, w3, b3, w4, b4)


def _qkv_body(nf, wq, wk, wv, qo, ko, vo):
    x = nf[...]
    qo[...] = jnp.dot(x, wq[...], preferred_element_type=jnp.float32)
    ko[...] = jnp.dot(x, wk[...], preferred_element_type=jnp.float32)
    vo[...] = jnp.dot(x, wv[...], preferred_element_type=jnp.float32)


def _qkv(nf, wq, wk, wv):
    BN = 1000
    grid = N // BN
    full = lambda: pl.BlockSpec((D, D), lambda i: (0, 0))
    s = jax.ShapeDtypeStruct((N, D), jnp.float32)
    return pl.pallas_call(
        _qkv_body,
        grid=(grid,),
        in_specs=[pl.BlockSpec((BN, D), lambda i: (i, 0)), full(), full(), full()],
        out_specs=[pl.BlockSpec((BN, D), lambda i: (i, 0))] * 3,
        out_shape=[s, s, s],
    )(nf, wq, wk, wv)


def _sc_body(wp, qt, kt, vt, snd, rcv, cut, out,
             sidx0, ridx0, cutv0, wv0, qv0, kv0, vv0,
             sidx1, ridx1, cutv1, wv1, qv1, kv1, vv1,
             c_v, acc, semi0, semg0, semi1, semg1):
    c = lax.axis_index("c")
    s = lax.axis_index("s")
    wid = c * NS + s
    base0 = wid * EPW

    sets = ((sidx0, ridx0, cutv0, wv0, qv0, kv0, vv0, semi0, semg0),
            (sidx1, ridx1, cutv1, wv1, qv1, kv1, vv1, semi1, semg1))

    def zrow(r, carry):
        for j in range(D // 16):
            c_v[r, pl.ds(j * 16, 16)] = jnp.zeros((16,), jnp.float32)
        return carry
    lax.fori_loop(0, ZC, zrow, 0)

    def zchunk(j, carry):
        i = s + j * NS
        @pl.when(i < NZFULL)
        def _():
            pltpu.sync_copy(c_v, acc.at[pl.ds(i * ZC, ZC)])
        return carry
    lax.fori_loop(0, (NZFULL + NS - 1) // NS, zchunk, 0)

    @pl.when(s == 0)
    def _():
        pltpu.sync_copy(c_v.at[pl.ds(0, ZREM)], acc.at[pl.ds(NZFULL * ZC, ZREM)])
    plsc.subcore_barrier()

    def issue_idx(i, st):
        sidx, ridx, cutv = st[0], st[1], st[2]
        semi = st[7]
        b = base0 + i * BE
        pltpu.async_copy(snd.at[pl.ds(b, BE)], sidx, semi)
        pltpu.async_copy(rcv.at[pl.ds(b, BE)], ridx, semi)
        pltpu.async_copy(cut.at[pl.ds(b, BE)], cutv.at[pl.ds(0, BE)], semi)

    def wait_idx(st):
        sidx, ridx, cutv = st[0], st[1], st[2]
        semi = st[7]
        pltpu.make_async_copy(snd.at[pl.ds(0, BE)], sidx, semi).wait()
        pltpu.make_async_copy(rcv.at[pl.ds(0, BE)], ridx, semi).wait()
        pltpu.make_async_copy(cut.at[pl.ds(0, BE)], cutv.at[pl.ds(0, BE)], semi).wait()

    def issue_gathers(i, st):
        sidx, ridx, wv, qv, kv, vv = st[0], st[1], st[3], st[4], st[5], st[6]
        semg = st[8]
        b = base0 + i * BE
        pltpu.async_copy(wp.at[pl.ds(b, BE)], wv, semg)
        pltpu.async_copy(qt.at[ridx], qv, semg)
        pltpu.async_copy(kt.at[sidx], kv, semg)
        pltpu.async_copy(vt.at[sidx], vv, semg)

    def wait_gathers(st):
        semg = st[8]
        for dst in (st[3], st[4], st[5], st[6]):
            pltpu.make_async_copy(wp.at[pl.ds(0, BE)], dst, semg).wait()

    lanes = lax.iota(jnp.int32, 16)
    perms = [lanes ^ jnp.int32(1 << jj) for jj in (3, 2, 1, 0)]
    dnums = lax.GatherDimensionNumbers(
        offset_dims=(), collapsed_slice_dims=(0,), start_index_map=(0,))

    def lane_perm(x, pm):
        return lax.gather(x, pm[:, None], dnums, slice_sizes=(1,),
                          mode=lax.GatherScatterMode.PROMISE_IN_BOUNDS)

    def compute_block(st):
        ridx, cutv, wv, qv, kv, vv = st[1], st[2], st[3], st[4], st[5], st[6]

        @plsc.parallel_loop(0, BE, 1, unroll=4)
        def edge(e):
            wbase = e & ~jnp.int32(15)
            lane = e & jnp.int32(15)
            cvec = cutv[pl.ds(wbase, 16)]
            scale = lane_perm(cvec, jnp.full((16,), 0, jnp.int32) + lane) * SCALE
            for h in range(H):
                dsl = pl.ds(h * HD, HD)
                p = qv[e, dsl] * wv[e, dsl] * kv[e, dsl]
                sv = scale * vv[e, dsl]
                for pm in perms:
                    p = p + lane_perm(p, pm)
                c_v[e, dsl] = p * sv
        pltpu.sync_copy(c_v.at[pl.ds(0, BE)], acc.at[ridx], add=True)

    issue_idx(0, sets[0])
    wait_idx(sets[0])
    issue_gathers(0, sets[0])
    issue_idx(1, sets[1])

    def pair(t, carry):
        for par in (0, 1):
            i = 2 * t + par
            cur, nxt = sets[par], sets[1 - par]

            @pl.when(i + 1 < NBPW)
            def _():
                wait_idx(nxt)
                issue_gathers(i + 1, nxt)
            wait_gathers(cur)
            compute_block(cur)

            @pl.when(i + 2 < NBPW)
            def _():
                issue_idx(i + 2, cur)
        return carry
    lax.fori_loop(0, NBPW // 2, pair, 0)

    plsc.subcore_barrier()

    def ochunk(j, carry):
        i = s + j * NS
        @pl.when(i < NZFULL)
        def _():
            r0 = i * ZC
            pltpu.sync_copy(acc.at[pl.ds(r0, ZC)], c_v)
            pltpu.sync_copy(c_v, out.at[c, pl.ds(r0, ZC)])
        return carry
    lax.fori_loop(0, (NZFULL + NS - 1) // NS, ochunk, 0)

    @pl.when(s == 0)
    def _():
        r0 = NZFULL * ZC
        pltpu.sync_copy(acc.at[pl.ds(r0, ZREM)], c_v.at[pl.ds(0, ZREM)])
        pltpu.sync_copy(c_v.at[pl.ds(0, ZREM)], out.at[c, pl.ds(r0, ZREM)])


def _sc_scatter(wp, qt, kt, vt, snd, rcv, cut):
    mesh = plsc.VectorSubcoreMesh(core_axis_name="c", subcore_axis_name="s")
    buf = lambda: [
        pltpu.VMEM((BE,), jnp.int32),
        pltpu.VMEM((BE,), jnp.int32),
        pltpu.VMEM((BE + 16,), jnp.float32),
        pltpu.VMEM((BE, D), jnp.float32),
        pltpu.VMEM((BE, D), jnp.float32),
        pltpu.VMEM((BE, D), jnp.float32),
        pltpu.VMEM((BE, D), jnp.float32),
    ]
    f = functools.partial(
        pl.kernel,
        out_type=jax.ShapeDtypeStruct((NC, N, D), jnp.float32),
        mesh=mesh,
        scratch_types=buf() + buf() + [
            pltpu.VMEM((ZC, D), jnp.float32),
            pltpu.VMEM_SHARED((N, D), jnp.float32),
            pltpu.SemaphoreType.DMA,
            pltpu.SemaphoreType.DMA,
            pltpu.SemaphoreType.DMA,
            pltpu.SemaphoreType.DMA,
        ],
    )(_sc_body)
    return f(wp, qt, kt, vt, snd, rcv, cut)


def _sum_body(p, o):
    o[...] = p[0] + p[1]


def _sum_partials(parts):
    BN = 1000
    return pl.pallas_call(
        _sum_body,
        grid=(N // BN,),
        in_specs=[pl.BlockSpec((NC, BN, D), lambda i: (0, i, 0))],
        out_specs=pl.BlockSpec((BN, D), lambda i: (i, 0)),
        out_shape=jax.ShapeDtypeStruct((N, D), jnp.float32),
    )(parts)


def kernel(node_feats, edge_feats, chi_scalar, cutoffs, senders, receivers,
           W_rad1, b_rad1, W_rad2, b_rad2,
           W_sph1, b_sph1, W_sph2, b_sph2,
           Wq, Wk, Wv):
    wp = _edge_mlp(edge_feats, chi_scalar,
                   W_rad1, b_rad1.reshape(1, 64), W_rad2, b_rad2.reshape(1, D),
                   W_sph1, b_sph1.reshape(1, 64), W_sph2, b_sph2.reshape(1, D))
    qt, kt, vt = _qkv(node_feats, Wq, Wk, Wv)
    parts = _sc_scatter(wp, qt, kt, vt,
                        senders.astype(jnp.int32), receivers.astype(jnp.int32),
                        cutoffs.astype(jnp.float32))
    return _sum_partials(parts)


# speedup vs baseline: 1.8281x; 1.0282x over previous
import functools
import jax
import jax.numpy as jnp
from jax import lax
from jax.experimental import pallas as pl
from jax.experimental.pallas import tpu as pltpu
from jax.experimental.pallas import tpu_sc as plsc

N = 10000
E = 320000
D = 128
H = 8
HD = 16

NC = 2
NS = 16
NW = NC * NS
EPW = E // NW
BE = 40
NBPW = EPW // BE
ZC = 64
NZFULL = N // ZC
ZREM = N - NZFULL * ZC
SCALE = 1.0 / (4.0 * 32.0)


def _edge_mlp_body(ef, chi, w1, b1, w2, b2, w3, b3, w4, b4, out):
    h1 = jnp.dot(ef[...], w1[...], preferred_element_type=jnp.float32) + b1[...]
    h1 = h1 * jax.nn.sigmoid(h1)
    r = jnp.dot(h1, w2[...], preferred_element_type=jnp.float32) + b2[...]
    h2 = jnp.dot(chi[...], w3[...], preferred_element_type=jnp.float32) + b3[...]
    h2 = h2 * jax.nn.sigmoid(h2)
    out[...] = r + jnp.dot(h2, w4[...], preferred_element_type=jnp.float32) + b4[...]


def _edge_mlp(ef, chi, w1, b1, w2, b2, w3, b3, w4, b4):
    BEW = 1600
    grid = E // BEW
    full = lambda shape: pl.BlockSpec(shape, lambda i: (0, 0))
    return pl.pallas_call(
        _edge_mlp_body,
        grid=(grid,),
        in_specs=[
            pl.BlockSpec((BEW, 16), lambda i: (i, 0)),
            pl.BlockSpec((BEW, 16), lambda i: (i, 0)),
            full((16, 64)), full((1, 64)),
            full((64, D)), full((1, D)),
            full((16, 64)), full((1, 64)),
            full((64, D)), full((1, D)),
        ],
        out_specs=pl.BlockSpec((BEW, D), lambda i: (i, 0)),
        out_shape=jax.ShapeDtypeStruct((E, D), jnp.float32),
    )(ef, chi, w1, b1, w2, b2, w3, b3, w4, b4)


def _qkv_body(nf, wq, wk, wv, qo, kvo):
    x = nf[...]
    qo[...] = jnp.dot(x, wq[...], preferred_element_type=jnp.float32)
    kvo[:, :D] = jnp.dot(x, wk[...], preferred_element_type=jnp.float32)
    kvo[:, D:] = jnp.dot(x, wv[...], preferred_element_type=jnp.float32)


def _qkv(nf, wq, wk, wv):
    BN = 1000
    grid = N // BN
    full = lambda: pl.BlockSpec((D, D), lambda i: (0, 0))
    return pl.pallas_call(
        _qkv_body,
        grid=(grid,),
        in_specs=[pl.BlockSpec((BN, D), lambda i: (i, 0)), full(), full(), full()],
        out_specs=[pl.BlockSpec((BN, D), lambda i: (i, 0)),
                   pl.BlockSpec((BN, 2 * D), lambda i: (i, 0))],
        out_shape=[jax.ShapeDtypeStruct((N, D), jnp.float32),
                   jax.ShapeDtypeStruct((N, 2 * D), jnp.float32)],
    )(nf, wq, wk, wv)


def _sc_body(wp, qt, kvt, snd, rcv, cut, out,
             sidx0, ridx0, cutv0, wv0, qv0, kvv0,
             sidx1, ridx1, cutv1, wv1, qv1, kvv1,
             c_v, acc, semi0, semg0, semi1, semg1):
    c = lax.axis_index("c")
    s = lax.axis_index("s")
    wid = c * NS + s
    base0 = wid * EPW

    sets = ((sidx0, ridx0, cutv0, wv0, qv0, kvv0, semi0, semg0),
            (sidx1, ridx1, cutv1, wv1, qv1, kvv1, semi1, semg1))

    def zrow(r, carry):
        for j in range(D // 16):
            c_v[r, pl.ds(j * 16, 16)] = jnp.zeros((16,), jnp.float32)
        return carry
    lax.fori_loop(0, ZC, zrow, 0)

    def zchunk(j, carry):
        i = s + j * NS
        @pl.when(i < NZFULL)
        def _():
            pltpu.sync_copy(c_v, acc.at[pl.ds(i * ZC, ZC)])
        return carry
    lax.fori_loop(0, (NZFULL + NS - 1) // NS, zchunk, 0)

    @pl.when(s == 0)
    def _():
        pltpu.sync_copy(c_v.at[pl.ds(0, ZREM)], acc.at[pl.ds(NZFULL * ZC, ZREM)])
    plsc.subcore_barrier()

    def issue_idx(i, st):
        sidx, ridx, cutv = st[0], st[1], st[2]
        semi = st[6]
        b = base0 + i * BE
        pltpu.async_copy(snd.at[pl.ds(b, BE)], sidx, semi)
        pltpu.async_copy(rcv.at[pl.ds(b, BE)], ridx, semi)
        pltpu.async_copy(cut.at[pl.ds(b, BE)], cutv.at[pl.ds(0, BE)], semi)

    def wait_idx(st):
        sidx, ridx, cutv = st[0], st[1], st[2]
        semi = st[6]
        pltpu.make_async_copy(snd.at[pl.ds(0, BE)], sidx, semi).wait()
        pltpu.make_async_copy(rcv.at[pl.ds(0, BE)], ridx, semi).wait()
        pltpu.make_async_copy(cut.at[pl.ds(0, BE)], cutv.at[pl.ds(0, BE)], semi).wait()

    def issue_gathers(i, st):
        sidx, ridx, wv, qv, kvv = st[0], st[1], st[3], st[4], st[5]
        semg = st[7]
        b = base0 + i * BE
        pltpu.async_copy(wp.at[pl.ds(b, BE)], wv, semg)
        pltpu.async_copy(qt.at[ridx], qv, semg)
        pltpu.async_copy(kvt.at[sidx], kvv, semg)

    def wait_gathers(st):
        semg = st[7]
        pltpu.make_async_copy(wp.at[pl.ds(0, BE)], st[3], semg).wait()
        pltpu.make_async_copy(wp.at[pl.ds(0, BE)], st[4], semg).wait()
        pltpu.make_async_copy(kvt.at[pl.ds(0, BE)], st[5], semg).wait()

    lanes = lax.iota(jnp.int32, 16)
    perms = [lanes ^ jnp.int32(1 << jj) for jj in (3, 2, 1, 0)]
    dnums = lax.GatherDimensionNumbers(
        offset_dims=(), collapsed_slice_dims=(0,), start_index_map=(0,))

    def lane_perm(x, pm):
        return lax.gather(x, pm[:, None], dnums, slice_sizes=(1,),
                          mode=lax.GatherScatterMode.PROMISE_IN_BOUNDS)

    def compute_block(st):
        ridx, cutv, wv, qv, kvv = st[1], st[2], st[3], st[4], st[5]

        @plsc.parallel_loop(0, BE, 1, unroll=8)
        def edge(e):
            wbase = e & ~jnp.int32(15)
            lane = e & jnp.int32(15)
            cvec = cutv[pl.ds(wbase, 16)]
            scale = lane_perm(cvec, jnp.full((16,), 0, jnp.int32) + lane) * SCALE
            for h in range(H):
                dsl = pl.ds(h * HD, HD)
                p = qv[e, dsl] * wv[e, dsl] * kvv[e, dsl]
                sv = scale * kvv[e, pl.ds(D + h * HD, HD)]
                for pm in perms:
                    p = p + lane_perm(p, pm)
                c_v[e, dsl] = p * sv
        pltpu.sync_copy(c_v.at[pl.ds(0, BE)], acc.at[ridx], add=True)

    issue_idx(0, sets[0])
    wait_idx(sets[0])
    issue_gathers(0, sets[0])
    issue_idx(1, sets[1])

    def pair(t, carry):
        for par in (0, 1):
            i = 2 * t + par
            cur, nxt = sets[par], sets[1 - par]

            @pl.when(i + 1 < NBPW)
            def _():
                wait_idx(nxt)
                issue_gathers(i + 1, nxt)
            wait_gathers(cur)
            compute_block(cur)

            @pl.when(i + 2 < NBPW)
            def _():
                issue_idx(i + 2, cur)
        return carry
    lax.fori_loop(0, NBPW // 2, pair, 0)

    plsc.subcore_barrier()

    def ochunk(j, carry):
        i = s + j * NS
        @pl.when(i < NZFULL)
        def _():
            r0 = i * ZC
            pltpu.sync_copy(acc.at[pl.ds(r0, ZC)], c_v)
            pltpu.sync_copy(c_v, out.at[c, pl.ds(r0, ZC)])
        return carry
    lax.fori_loop(0, (NZFULL + NS - 1) // NS, ochunk, 0)

    @pl.when(s == 0)
    def _():
        r0 = NZFULL * ZC
        pltpu.sync_copy(acc.at[pl.ds(r0, ZREM)], c_v.at[pl.ds(0, ZREM)])
        pltpu.sync_copy(c_v.at[pl.ds(0, ZREM)], out.at[c, pl.ds(r0, ZREM)])


def _sc_scatter(wp, qt, kvt, snd, rcv, cut):
    mesh = plsc.VectorSubcoreMesh(core_axis_name="c", subcore_axis_name="s")
    buf = lambda: [
        pltpu.VMEM((BE,), jnp.int32),
        pltpu.VMEM((BE,), jnp.int32),
        pltpu.VMEM((BE + 16,), jnp.float32),
        pltpu.VMEM((BE, D), jnp.float32),
        pltpu.VMEM((BE, D), jnp.float32),
        pltpu.VMEM((BE, 2 * D), jnp.float32),
    ]
    f = functools.partial(
        pl.kernel,
        out_type=jax.ShapeDtypeStruct((NC, N, D), jnp.float32),
        mesh=mesh,
        scratch_types=buf() + buf() + [
            pltpu.VMEM((ZC, D), jnp.float32),
            pltpu.VMEM_SHARED((N, D), jnp.float32),
            pltpu.SemaphoreType.DMA,
            pltpu.SemaphoreType.DMA,
            pltpu.SemaphoreType.DMA,
            pltpu.SemaphoreType.DMA,
        ],
    )(_sc_body)
    return f(wp, qt, kvt, snd, rcv, cut)


def _sum_body(p, o):
    o[...] = p[0] + p[1]


def _sum_partials(parts):
    BN = 1000
    return pl.pallas_call(
        _sum_body,
        grid=(N // BN,),
        in_specs=[pl.BlockSpec((NC, BN, D), lambda i: (0, i, 0))],
        out_specs=pl.BlockSpec((BN, D), lambda i: (i, 0)),
        out_shape=jax.ShapeDtypeStruct((N, D), jnp.float32),
    )(parts)


def kernel(node_feats, edge_feats, chi_scalar, cutoffs, senders, receivers,
           W_rad1, b_rad1, W_rad2, b_rad2,
           W_sph1, b_sph1, W_sph2, b_sph2,
           Wq, Wk, Wv):
    wp = _edge_mlp(edge_feats, chi_scalar,
                   W_rad1, b_rad1.reshape(1, 64), W_rad2, b_rad2.reshape(1, D),
                   W_sph1, b_sph1.reshape(1, 64), W_sph2, b_sph2.reshape(1, D))
    qt, kvt = _qkv(node_feats, Wq, Wk, Wv)
    parts = _sc_scatter(wp, qt, kvt,
                        _to_i32(senders), _to_i32(receivers),
                        cutoffs.astype(jnp.float32))
    return _sum_partials(parts)


def _to_i32(x):
    if x.dtype == jnp.int64:
        return lax.bitcast_convert_type(x, jnp.int32)[..., 0]
    return x.astype(jnp.int32)


# speedup vs baseline: 1.9145x; 1.0473x over previous
import functools
import jax
import jax.numpy as jnp
import numpy as np
from jax import lax
from jax.experimental import pallas as pl
from jax.experimental.pallas import tpu as pltpu
from jax.experimental.pallas import tpu_sc as plsc

N = 10000
E = 320000
D = 128
H = 8
HD = 16

NC = 2
NS = 16
NW = NC * NS
EPW = E // NW
BE = 40
NBPW = EPW // BE
ZC = 40
NZ = N // ZC
SCALE = 1.0 / (4.0 * 32.0)


def _pack2(a, b):
    au = lax.bitcast_convert_type(a.astype(jnp.bfloat16), jnp.uint16).astype(jnp.uint32)
    bu = lax.bitcast_convert_type(b.astype(jnp.bfloat16), jnp.uint16).astype(jnp.uint32)
    return au | (bu << 16)


def _edge_mlp_body(ef, chi, w1, b1, w2, b2, w3, b3, w4, b4, out):
    h1 = jnp.dot(ef[...], w1[...], preferred_element_type=jnp.float32) + b1[...]
    h1 = h1 * jax.nn.sigmoid(h1)
    r = jnp.dot(h1, w2[...], preferred_element_type=jnp.float32) + b2[...]
    h2 = jnp.dot(chi[...], w3[...], preferred_element_type=jnp.float32) + b3[...]
    h2 = h2 * jax.nn.sigmoid(h2)
    out[...] = r + jnp.dot(h2, w4[...], preferred_element_type=jnp.float32) + b4[...]


def _edge_mlp(ef, chi, w1, b1, w2, b2, w3, b3, w4, b4):
    BEW = 1600
    grid = E // BEW
    full = lambda shape: pl.BlockSpec(shape, lambda i: (0, 0))
    return pl.pallas_call(
        _edge_mlp_body,
        grid=(grid,),
        in_specs=[
            pl.BlockSpec((BEW, 16), lambda i: (i, 0)),
            pl.BlockSpec((BEW, 16), lambda i: (i, 0)),
            full((16, 64)), full((1, 64)),
            full((64, D)), full((1, D)),
            full((16, 64)), full((1, 64)),
            full((64, D)), full((1, D)),
        ],
        out_specs=pl.BlockSpec((BEW, D), lambda i: (i, 0)),
        out_shape=jax.ShapeDtypeStruct((E, D), jnp.float32),
    )(ef, chi, w1, b1, w2, b2, w3, b3, w4, b4)


def _qkv_body(nf, wq, wk, wv, qo, kvo):
    x = nf[...]
    qo[...] = jnp.dot(x, wq[...], preferred_element_type=jnp.float32)
    k = jnp.dot(x, wk[...], preferred_element_type=jnp.float32)
    v = jnp.dot(x, wv[...], preferred_element_type=jnp.float32)
    kvo[...] = _pack2(k, v)


def _qkv(nf, wq, wk, wv):
    BN = 1000
    grid = N // BN
    full = lambda: pl.BlockSpec((D, D), lambda i: (0, 0))
    return pl.pallas_call(
        _qkv_body,
        grid=(grid,),
        in_specs=[pl.BlockSpec((BN, D), lambda i: (i, 0)), full(), full(), full()],
        out_specs=[pl.BlockSpec((BN, D), lambda i: (i, 0)),
                   pl.BlockSpec((BN, D), lambda i: (i, 0))],
        out_shape=[jax.ShapeDtypeStruct((N, D), jnp.float32),
                   jax.ShapeDtypeStruct((N, D), jnp.uint32)],
    )(nf, wq, wk, wv)


def _sc_body(wp, qt, kvt, snd, rcv, cut, out,
             sidx0, ridx0, cutv0, qv0, kvv0,
             sidx1, ridx1, cutv1, qv1, kvv1,
             sidx2, ridx2, cutv2, qv2, kvv2,
             wv0, wv1, c_v, acc,
             semi0, semg0, semi1, semg1, semi2, semg2, semw0, semw1):
    c = lax.axis_index("c")
    s = lax.axis_index("s")
    wid = c * NS + s
    base0 = wid * EPW

    sets = ((sidx0, ridx0, cutv0, qv0, kvv0, semi0, semg0),
            (sidx1, ridx1, cutv1, qv1, kvv1, semi1, semg1),
            (sidx2, ridx2, cutv2, qv2, kvv2, semi2, semg2))
    wbufs = ((wv0, semw0), (wv1, semw1))

    def zrow(r, carry):
        for j in range(D // 16):
            c_v[r, pl.ds(j * 16, 16)] = jnp.zeros((16,), jnp.float32)
        return carry
    lax.fori_loop(0, ZC, zrow, 0)

    def zchunk(j, carry):
        i = s + j * NS
        @pl.when(i < NZ)
        def _():
            pltpu.sync_copy(c_v, acc.at[pl.ds(i * ZC, ZC)])
        return carry
    lax.fori_loop(0, (NZ + NS - 1) // NS, zchunk, 0)
    plsc.subcore_barrier()

    def issue_idx(i, st):
        b = base0 + i * BE
        pltpu.async_copy(snd.at[pl.ds(b, BE)], st[0], st[5])
        pltpu.async_copy(rcv.at[pl.ds(b, BE)], st[1], st[5])
        pltpu.async_copy(cut.at[pl.ds(b, BE)], st[2].at[pl.ds(0, BE)], st[5])

    def wait_idx(st):
        pltpu.make_async_copy(snd.at[pl.ds(0, BE)], st[0], st[5]).wait()
        pltpu.make_async_copy(rcv.at[pl.ds(0, BE)], st[1], st[5]).wait()
        pltpu.make_async_copy(cut.at[pl.ds(0, BE)], st[2].at[pl.ds(0, BE)], st[5]).wait()

    def issue_qkv(i, st):
        pltpu.async_copy(qt.at[st[1]], st[3], st[6])
        pltpu.async_copy(kvt.at[st[0]], st[4], st[6])

    def wait_qkv(st):
        pltpu.make_async_copy(qt.at[pl.ds(0, BE)], st[3], st[6]).wait()
        pltpu.make_async_copy(kvt.at[pl.ds(0, BE)], st[4], st[6]).wait()

    def issue_w(i, wb):
        pltpu.async_copy(wp.at[pl.ds(base0 + i * BE, BE)], wb[0], wb[1])

    def wait_w(wb):
        pltpu.make_async_copy(wp.at[pl.ds(0, BE)], wb[0], wb[1]).wait()

    lanes = lax.iota(jnp.int32, 16)
    perms = [lanes ^ jnp.int32(1 << jj) for jj in (3, 2, 1, 0)]
    dnums = lax.GatherDimensionNumbers(
        offset_dims=(), collapsed_slice_dims=(0,), start_index_map=(0,))

    def lane_perm(x, pm):
        return lax.gather(x, pm[:, None], dnums, slice_sizes=(1,),
                          mode=lax.GatherScatterMode.PROMISE_IN_BOUNDS)

    def compute_block(st, wb):
        ridx, cutv, qv, kvv = st[1], st[2], st[3], st[4]
        wv = wb[0]

        def unp(x):
            a = plsc.bitcast(x << jnp.uint32(16), jnp.float32)
            b = plsc.bitcast(x & jnp.uint32(0xFFFF0000), jnp.float32)
            return a, b

        @plsc.parallel_loop(0, BE, 1, unroll=8)
        def edge(e):
            wbase = e & ~jnp.int32(15)
            lane = e & jnp.int32(15)
            cvec = cutv[pl.ds(wbase, 16)]
            scale = lane_perm(cvec, jnp.full((16,), 0, jnp.int32) + lane) * SCALE
            for h in range(H):
                dsl = pl.ds(h * HD, HD)
                kh, vh = unp(kvv[e, dsl])
                p = qv[e, dsl] * wv[e, dsl] * kh
                sv = scale * vh
                for pm in perms:
                    p = p + lane_perm(p, pm)
                c_v[e, dsl] = p * sv
        pltpu.sync_copy(c_v.at[pl.ds(0, BE)], acc.at[ridx], add=True)

    issue_idx(0, sets[0])
    issue_idx(1, sets[1])
    issue_idx(2, sets[2])
    issue_w(0, wbufs[0])
    wait_idx(sets[0])
    issue_qkv(0, sets[0])
    wait_idx(sets[1])
    issue_qkv(1, sets[1])

    def six(t, carry):
        for u in range(6):
            i6 = 6 * t + u
            cur = sets[u % 3]
            wb = wbufs[u % 2]

            @pl.when(i6 + 2 < NBPW)
            def _():
                wait_idx(sets[(u + 2) % 3])
                issue_qkv(i6 + 2, sets[(u + 2) % 3])

            @pl.when(i6 + 1 < NBPW)
            def _():
                issue_w(i6 + 1, wbufs[(u + 1) % 2])

            @pl.when(i6 < NBPW)
            def _():
                wait_qkv(cur)
                wait_w(wb)
                compute_block(cur, wb)

            @pl.when(i6 + 3 < NBPW)
            def _():
                issue_idx(i6 + 3, cur)
        return carry
    lax.fori_loop(0, (NBPW + 5) // 6, six, 0)

    plsc.subcore_barrier()

    def ochunk(j, carry):
        i = s + j * NS
        @pl.when(i < NZ)
        def _():
            r0 = i * ZC
            pltpu.sync_copy(acc.at[pl.ds(r0, ZC)], c_v)
            pltpu.sync_copy(c_v, out.at[c, pl.ds(r0, ZC)])
        return carry
    lax.fori_loop(0, (NZ + NS - 1) // NS, ochunk, 0)


def _sc_scatter(wp, qt, kvt, snd, rcv, cut):
    mesh = plsc.VectorSubcoreMesh(core_axis_name="c", subcore_axis_name="s")
    buf = lambda: [
        pltpu.VMEM((BE,), jnp.int32),
        pltpu.VMEM((BE,), jnp.int32),
        pltpu.VMEM((BE + 8,), jnp.float32),
        pltpu.VMEM((BE, D), jnp.float32),
        pltpu.VMEM((BE, D), jnp.uint32),
    ]
    f = functools.partial(
        pl.kernel,
        out_type=jax.ShapeDtypeStruct((NC, N, D), jnp.float32),
        mesh=mesh,
        compiler_params=pltpu.CompilerParams(needs_layout_passes=False),
        scratch_types=buf() + buf() + buf() + [
            pltpu.VMEM((BE, D), jnp.float32),
            pltpu.VMEM((BE, D), jnp.float32),
            pltpu.VMEM((ZC, D), jnp.float32),
            pltpu.VMEM_SHARED((N, D), jnp.float32),
            pltpu.SemaphoreType.DMA,
            pltpu.SemaphoreType.DMA,
            pltpu.SemaphoreType.DMA,
            pltpu.SemaphoreType.DMA,
            pltpu.SemaphoreType.DMA,
            pltpu.SemaphoreType.DMA,
            pltpu.SemaphoreType.DMA,
            pltpu.SemaphoreType.DMA,
        ],
    )(_sc_body)
    return f(wp, qt, kvt, snd, rcv, cut)


def _sum_body(p, o):
    o[...] = p[0] + p[1]


def _sum_partials(parts):
    BN = 1000
    return pl.pallas_call(
        _sum_body,
        grid=(N // BN,),
        in_specs=[pl.BlockSpec((NC, BN, D), lambda i: (0, i, 0))],
        out_specs=pl.BlockSpec((BN, D), lambda i: (i, 0)),
        out_shape=jax.ShapeDtypeStruct((N, D), jnp.float32),
    )(parts)


def kernel(node_feats, edge_feats, chi_scalar, cutoffs, senders, receivers,
           W_rad1, b_rad1, W_rad2, b_rad2,
           W_sph1, b_sph1, W_sph2, b_sph2,
           Wq, Wk, Wv):
    wp = _edge_mlp(edge_feats, chi_scalar,
                   W_rad1, b_rad1.reshape(1, 64), W_rad2, b_rad2.reshape(1, D),
                   W_sph1, b_sph1.reshape(1, 64), W_sph2, b_sph2.reshape(1, D))
    qt, kvt = _qkv(node_feats, Wq, Wk, Wv)
    parts = _sc_scatter(wp, qt, kvt,
                        _to_i32(senders), _to_i32(receivers),
                        cutoffs.astype(jnp.float32))
    return _sum_partials(parts)


def _to_i32(x):
    if x.dtype == jnp.int64:
        return lax.bitcast_convert_type(x, jnp.int32)[..., 0]
    return x.astype(jnp.int32)


# speedup vs baseline: 2.2066x; 1.1526x over previous
import functools
import jax
import jax.numpy as jnp
import numpy as np
from jax import lax
from jax.experimental import pallas as pl
from jax.experimental.pallas import tpu as pltpu
from jax.experimental.pallas import tpu_sc as plsc

N = 10000
E = 320000
D = 128
H = 8
HD = 16

NC = 2
NS = 16
NW = NC * NS
EPW = E // NW
BE = 40
NBPW = EPW // BE
ZC = 40
NZ = N // ZC
SCALE = 1.0 / (4.0 * 32.0)


def _pack2(a, b):
    au = lax.bitcast_convert_type(a.astype(jnp.bfloat16), jnp.uint16).astype(jnp.uint32)
    bu = lax.bitcast_convert_type(b.astype(jnp.bfloat16), jnp.uint16).astype(jnp.uint32)
    return au | (bu << 16)


def _edge_mlp_body(ef, chi, w1, b1, w2, b2, w3, b3, w4, b4, out):
    h1 = jnp.dot(ef[...], w1[...], preferred_element_type=jnp.float32) + b1[...]
    h1 = h1 * jax.nn.sigmoid(h1)
    r = jnp.dot(h1, w2[...], preferred_element_type=jnp.float32) + b2[...]
    h2 = jnp.dot(chi[...], w3[...], preferred_element_type=jnp.float32) + b3[...]
    h2 = h2 * jax.nn.sigmoid(h2)
    out[...] = r + jnp.dot(h2, w4[...], preferred_element_type=jnp.float32) + b4[...]


def _edge_mlp(ef, chi, w1, b1, w2, b2, w3, b3, w4, b4):
    BEW = 1600
    grid = E // BEW
    full = lambda shape: pl.BlockSpec(shape, lambda i: (0, 0))
    return pl.pallas_call(
        _edge_mlp_body,
        grid=(grid,),
        in_specs=[
            pl.BlockSpec((BEW, 16), lambda i: (i, 0)),
            pl.BlockSpec((BEW, 16), lambda i: (i, 0)),
            full((16, 64)), full((1, 64)),
            full((64, D)), full((1, D)),
            full((16, 64)), full((1, 64)),
            full((64, D)), full((1, D)),
        ],
        out_specs=pl.BlockSpec((BEW, D), lambda i: (i, 0)),
        out_shape=jax.ShapeDtypeStruct((E, D), jnp.float32),
    )(ef, chi, w1, b1, w2, b2, w3, b3, w4, b4)


def _qkv_body(nf, wq, wk, wv, qo, kvo):
    x = nf[...]
    qo[...] = jnp.dot(x, wq[...], preferred_element_type=jnp.float32)
    k = jnp.dot(x, wk[...], preferred_element_type=jnp.float32)
    v = jnp.dot(x, wv[...], preferred_element_type=jnp.float32)
    kvo[...] = _pack2(k, v)


def _qkv(nf, wq, wk, wv):
    BN = 1000
    grid = N // BN
    full = lambda: pl.BlockSpec((D, D), lambda i: (0, 0))
    return pl.pallas_call(
        _qkv_body,
        grid=(grid,),
        in_specs=[pl.BlockSpec((BN, D), lambda i: (i, 0)), full(), full(), full()],
        out_specs=[pl.BlockSpec((BN, D), lambda i: (i, 0)),
                   pl.BlockSpec((BN, D), lambda i: (i, 0))],
        out_shape=[jax.ShapeDtypeStruct((N, D), jnp.float32),
                   jax.ShapeDtypeStruct((N, D), jnp.uint32)],
    )(nf, wq, wk, wv)


def _sc_body(wp, qt, kvt, snd, rcv, cut, out,
             sidx0, ridx0, cutv0, qv0, kvv0,
             sidx1, ridx1, cutv1, qv1, kvv1,
             sidx2, ridx2, cutv2, qv2, kvv2,
             wv0, wv1, c_v, acc,
             semi0, semg0, semi1, semg1, semi2, semg2, semw0, semw1, sem_sc):
    c = lax.axis_index("c")
    s = lax.axis_index("s")
    wid = c * NS + s
    base0 = wid * EPW

    sets = ((sidx0, ridx0, cutv0, qv0, kvv0, semi0, semg0),
            (sidx1, ridx1, cutv1, qv1, kvv1, semi1, semg1),
            (sidx2, ridx2, cutv2, qv2, kvv2, semi2, semg2))
    wbufs = ((wv0, semw0), (wv1, semw1))

    def zrow(r, carry):
        for j in range(D // 16):
            c_v[r, pl.ds(j * 16, 16)] = jnp.zeros((16,), jnp.float32)
        return carry
    lax.fori_loop(0, ZC, zrow, 0)

    def zchunk(j, carry):
        i = s + j * NS
        @pl.when(i < NZ)
        def _():
            pltpu.sync_copy(c_v, acc.at[pl.ds(i * ZC, ZC)])
        return carry
    lax.fori_loop(0, (NZ + NS - 1) // NS, zchunk, 0)
    plsc.subcore_barrier()

    def issue_idx(i, st):
        b = base0 + i * BE
        pltpu.async_copy(snd.at[pl.ds(b, BE)], st[0], st[5])
        pltpu.async_copy(rcv.at[pl.ds(b, BE)], st[1], st[5])
        pltpu.async_copy(cut.at[pl.ds(b, BE)], st[2].at[pl.ds(0, BE)], st[5])

    def wait_idx(st):
        pltpu.make_async_copy(snd.at[pl.ds(0, BE)], st[0], st[5]).wait()
        pltpu.make_async_copy(rcv.at[pl.ds(0, BE)], st[1], st[5]).wait()
        pltpu.make_async_copy(cut.at[pl.ds(0, BE)], st[2].at[pl.ds(0, BE)], st[5]).wait()

    def issue_qkv(i, st):
        pltpu.async_copy(qt.at[st[1]], st[3], st[6])
        pltpu.async_copy(kvt.at[st[0]], st[4], st[6])

    def wait_qkv(st):
        pltpu.make_async_copy(qt.at[pl.ds(0, BE)], st[3], st[6]).wait()
        pltpu.make_async_copy(kvt.at[pl.ds(0, BE)], st[4], st[6]).wait()

    def issue_w(i, wb):
        pltpu.async_copy(wp.at[pl.ds(base0 + i * BE, BE)], wb[0], wb[1])

    def wait_w(wb):
        pltpu.make_async_copy(wp.at[pl.ds(0, BE)], wb[0], wb[1]).wait()

    lanes = lax.iota(jnp.int32, 16)
    perms = [lanes ^ jnp.int32(1 << jj) for jj in (3, 2, 1, 0)]
    dnums = lax.GatherDimensionNumbers(
        offset_dims=(), collapsed_slice_dims=(0,), start_index_map=(0,))

    def lane_perm(x, pm):
        return lax.gather(x, pm[:, None], dnums, slice_sizes=(1,),
                          mode=lax.GatherScatterMode.PROMISE_IN_BOUNDS)

    def compute_block(st, wb):
        ridx, cutv, qv, kvv = st[1], st[2], st[3], st[4]
        wv = wb[0]

        def unp(x):
            a = plsc.bitcast(x << jnp.uint32(16), jnp.float32)
            b = plsc.bitcast(x & jnp.uint32(0xFFFF0000), jnp.float32)
            return a, b

        @plsc.parallel_loop(0, BE, 1, unroll=8)
        def edge(e):
            wbase = e & ~jnp.int32(15)
            lane = e & jnp.int32(15)
            cvec = cutv[pl.ds(wbase, 16)]
            scale = lane_perm(cvec, jnp.full((16,), 0, jnp.int32) + lane) * SCALE
            for h in range(H):
                dsl = pl.ds(h * HD, HD)
                kh, vh = unp(kvv[e, dsl])
                p = qv[e, dsl] * wv[e, dsl] * kh
                sv = scale * vh
                c_v[e, dsl] = jnp.sum(p) * sv
        pltpu.async_copy(c_v.at[pl.ds(0, BE)], acc.at[ridx], sem_sc, add=True)

    def drain_scatter():
        pltpu.make_async_copy(c_v.at[pl.ds(0, BE)], acc.at[pl.ds(0, BE)],
                              sem_sc).wait()

    issue_idx(0, sets[0])
    issue_idx(1, sets[1])
    issue_idx(2, sets[2])
    issue_w(0, wbufs[0])
    wait_idx(sets[0])
    issue_qkv(0, sets[0])
    wait_idx(sets[1])
    issue_qkv(1, sets[1])

    def six(t, carry):
        for u in range(6):
            i6 = 6 * t + u
            cur = sets[u % 3]
            wb = wbufs[u % 2]

            @pl.when(i6 + 2 < NBPW)
            def _():
                wait_idx(sets[(u + 2) % 3])
                issue_qkv(i6 + 2, sets[(u + 2) % 3])

            @pl.when(i6 + 1 < NBPW)
            def _():
                issue_w(i6 + 1, wbufs[(u + 1) % 2])

            @pl.when(i6 < NBPW)
            def _():
                wait_qkv(cur)
                wait_w(wb)
                @pl.when(i6 > 0)
                def _():
                    drain_scatter()
                compute_block(cur, wb)

            @pl.when(i6 + 3 < NBPW)
            def _():
                issue_idx(i6 + 3, cur)
        return carry
    lax.fori_loop(0, (NBPW + 5) // 6, six, 0)
    drain_scatter()

    plsc.subcore_barrier()

    def ochunk(j, carry):
        i = s + j * NS
        @pl.when(i < NZ)
        def _():
            r0 = i * ZC
            pltpu.sync_copy(acc.at[pl.ds(r0, ZC)], c_v)
            pltpu.sync_copy(c_v, out.at[c, pl.ds(r0, ZC)])
        return carry
    lax.fori_loop(0, (NZ + NS - 1) // NS, ochunk, 0)


def _sc_scatter(wp, qt, kvt, snd, rcv, cut):
    mesh = plsc.VectorSubcoreMesh(core_axis_name="c", subcore_axis_name="s")
    buf = lambda: [
        pltpu.VMEM((BE,), jnp.int32),
        pltpu.VMEM((BE,), jnp.int32),
        pltpu.VMEM((BE + 8,), jnp.float32),
        pltpu.VMEM((BE, D), jnp.float32),
        pltpu.VMEM((BE, D), jnp.uint32),
    ]
    f = functools.partial(
        pl.kernel,
        out_type=jax.ShapeDtypeStruct((NC, N, D), jnp.float32),
        mesh=mesh,
        compiler_params=pltpu.CompilerParams(needs_layout_passes=False),
        scratch_types=buf() + buf() + buf() + [
            pltpu.VMEM((BE, D), jnp.float32),
            pltpu.VMEM((BE, D), jnp.float32),
            pltpu.VMEM((ZC, D), jnp.float32),
            pltpu.VMEM_SHARED((N, D), jnp.float32),
            pltpu.SemaphoreType.DMA,
            pltpu.SemaphoreType.DMA,
            pltpu.SemaphoreType.DMA,
            pltpu.SemaphoreType.DMA,
            pltpu.SemaphoreType.DMA,
            pltpu.SemaphoreType.DMA,
            pltpu.SemaphoreType.DMA,
            pltpu.SemaphoreType.DMA,
            pltpu.SemaphoreType.DMA,
        ],
    )(_sc_body)
    return f(wp, qt, kvt, snd, rcv, cut)


def _sum_body(p, o):
    o[...] = p[0] + p[1]


def _sum_partials(parts):
    BN = 1000
    return pl.pallas_call(
        _sum_body,
        grid=(N // BN,),
        in_specs=[pl.BlockSpec((NC, BN, D), lambda i: (0, i, 0))],
        out_specs=pl.BlockSpec((BN, D), lambda i: (i, 0)),
        out_shape=jax.ShapeDtypeStruct((N, D), jnp.float32),
    )(parts)


def kernel(node_feats, edge_feats, chi_scalar, cutoffs, senders, receivers,
           W_rad1, b_rad1, W_rad2, b_rad2,
           W_sph1, b_sph1, W_sph2, b_sph2,
           Wq, Wk, Wv):
    wp = _edge_mlp(edge_feats, chi_scalar,
                   W_rad1, b_rad1.reshape(1, 64), W_rad2, b_rad2.reshape(1, D),
                   W_sph1, b_sph1.reshape(1, 64), W_sph2, b_sph2.reshape(1, D))
    qt, kvt = _qkv(node_feats, Wq, Wk, Wv)
    parts = _sc_scatter(wp, qt, kvt,
                        _to_i32(senders), _to_i32(receivers),
                        cutoffs.astype(jnp.float32))
    return _sum_partials(parts)


def _to_i32(x):
    if x.dtype == jnp.int64:
        return lax.bitcast_convert_type(x, jnp.int32)[..., 0]
    return x.astype(jnp.int32)


# speedup vs baseline: 2.3861x; 1.0814x over previous
import functools
import jax
import jax.numpy as jnp
import numpy as np
from jax import lax
from jax.experimental import pallas as pl
from jax.experimental.pallas import tpu as pltpu
from jax.experimental.pallas import tpu_sc as plsc

N = 10000
E = 320000
D = 128
H = 8
HD = 16

NC = 2
NS = 16
NW = NC * NS
EPW = E // NW
BE = 40
NBPW = EPW // BE
ZC = 40
NZ = N // ZC
SCALE = 1.0 / (4.0 * 32.0)


def _pack2(a, b):
    au = lax.bitcast_convert_type(a.astype(jnp.bfloat16), jnp.uint16).astype(jnp.uint32)
    bu = lax.bitcast_convert_type(b.astype(jnp.bfloat16), jnp.uint16).astype(jnp.uint32)
    return au | (bu << 16)


def _edge_mlp_body(ef, chi, w1, b1, w2, b2, w3, b3, w4, b4, out):
    h1 = jnp.dot(ef[...], w1[...], preferred_element_type=jnp.float32) + b1[...]
    h1 = h1 * jax.nn.sigmoid(h1)
    r = jnp.dot(h1, w2[...], preferred_element_type=jnp.float32) + b2[...]
    h2 = jnp.dot(chi[...], w3[...], preferred_element_type=jnp.float32) + b3[...]
    h2 = h2 * jax.nn.sigmoid(h2)
    out[...] = r + jnp.dot(h2, w4[...], preferred_element_type=jnp.float32) + b4[...]


def _edge_mlp(ef, chi, w1, b1, w2, b2, w3, b3, w4, b4):
    BEW = 3200
    grid = E // BEW
    full = lambda shape: pl.BlockSpec(shape, lambda i: (0, 0))
    return pl.pallas_call(
        _edge_mlp_body,
        grid=(grid,),
        in_specs=[
            pl.BlockSpec((BEW, 16), lambda i: (i, 0)),
            pl.BlockSpec((BEW, 16), lambda i: (i, 0)),
            full((16, 64)), full((1, 64)),
            full((64, D)), full((1, D)),
            full((16, 64)), full((1, 64)),
            full((64, D)), full((1, D)),
        ],
        out_specs=pl.BlockSpec((BEW, D), lambda i: (i, 0)),
        out_shape=jax.ShapeDtypeStruct((E, D), jnp.float32),
    )(ef, chi, w1, b1, w2, b2, w3, b3, w4, b4)


def _qkv_body(nf, wq, wk, wv, qo, kvo):
    x = nf[...]
    qo[...] = jnp.dot(x, wq[...], preferred_element_type=jnp.float32)
    k = jnp.dot(x, wk[...], preferred_element_type=jnp.float32)
    v = jnp.dot(x, wv[...], preferred_element_type=jnp.float32)
    kvo[...] = _pack2(k, v)


def _qkv(nf, wq, wk, wv):
    BN = 1000
    grid = N // BN
    full = lambda: pl.BlockSpec((D, D), lambda i: (0, 0))
    return pl.pallas_call(
        _qkv_body,
        grid=(grid,),
        in_specs=[pl.BlockSpec((BN, D), lambda i: (i, 0)), full(), full(), full()],
        out_specs=[pl.BlockSpec((BN, D), lambda i: (i, 0)),
                   pl.BlockSpec((BN, D), lambda i: (i, 0))],
        out_shape=[jax.ShapeDtypeStruct((N, D), jnp.float32),
                   jax.ShapeDtypeStruct((N, D), jnp.uint32)],
    )(nf, wq, wk, wv)


def _sc_body(wp, qt, kvt, snd, rcv, cut, out,
             sidx0, ridx0, cutv0, qv0, kvv0,
             sidx1, ridx1, cutv1, qv1, kvv1,
             sidx2, ridx2, cutv2, qv2, kvv2,
             wv0, wv1, c_v, acc,
             semi0, semg0, semi1, semg1, semi2, semg2, semw0, semw1, sem_sc):
    c = lax.axis_index("c")
    s = lax.axis_index("s")
    wid = c * NS + s
    base0 = wid * EPW

    sets = ((sidx0, ridx0, cutv0, qv0, kvv0, semi0, semg0),
            (sidx1, ridx1, cutv1, qv1, kvv1, semi1, semg1),
            (sidx2, ridx2, cutv2, qv2, kvv2, semi2, semg2))
    wbufs = ((wv0, semw0), (wv1, semw1))

    def zrow(r, carry):
        for j in range(D // 16):
            c_v[r, pl.ds(j * 16, 16)] = jnp.zeros((16,), jnp.float32)
        return carry
    lax.fori_loop(0, ZC, zrow, 0)

    def zchunk(j, carry):
        i = s + j * NS
        @pl.when(i < NZ)
        def _():
            pltpu.sync_copy(c_v, acc.at[pl.ds(i * ZC, ZC)])
        return carry
    lax.fori_loop(0, (NZ + NS - 1) // NS, zchunk, 0)
    plsc.subcore_barrier()

    def issue_idx(i, st):
        b = base0 + i * BE
        pltpu.async_copy(snd.at[pl.ds(b, BE)], st[0], st[5])
        pltpu.async_copy(rcv.at[pl.ds(b, BE)], st[1], st[5])
        pltpu.async_copy(cut.at[pl.ds(b, BE)], st[2].at[pl.ds(0, BE)], st[5])

    def wait_idx(st):
        pltpu.make_async_copy(snd.at[pl.ds(0, BE)], st[0], st[5]).wait()
        pltpu.make_async_copy(rcv.at[pl.ds(0, BE)], st[1], st[5]).wait()
        pltpu.make_async_copy(cut.at[pl.ds(0, BE)], st[2].at[pl.ds(0, BE)], st[5]).wait()

    def issue_qkv(i, st):
        pltpu.async_copy(qt.at[st[1]], st[3], st[6])
        pltpu.async_copy(kvt.at[st[0]], st[4], st[6])

    def wait_qkv(st):
        pltpu.make_async_copy(qt.at[pl.ds(0, BE)], st[3], st[6]).wait()
        pltpu.make_async_copy(kvt.at[pl.ds(0, BE)], st[4], st[6]).wait()

    def issue_w(i, wb):
        pltpu.async_copy(wp.at[pl.ds(base0 + i * BE, BE)], wb[0], wb[1])

    def wait_w(wb):
        pltpu.make_async_copy(wp.at[pl.ds(0, BE)], wb[0], wb[1]).wait()

    lanes = lax.iota(jnp.int32, 16)
    perms = [lanes ^ jnp.int32(1 << jj) for jj in (3, 2, 1, 0)]
    dnums = lax.GatherDimensionNumbers(
        offset_dims=(), collapsed_slice_dims=(0,), start_index_map=(0,))

    def lane_perm(x, pm):
        return lax.gather(x, pm[:, None], dnums, slice_sizes=(1,),
                          mode=lax.GatherScatterMode.PROMISE_IN_BOUNDS)

    def compute_block(st, wb):
        ridx, cutv, qv, kvv = st[1], st[2], st[3], st[4]
        wv = wb[0]

        def unp(x):
            a = plsc.bitcast(x << jnp.uint32(16), jnp.float32)
            b = plsc.bitcast(x & jnp.uint32(0xFFFF0000), jnp.float32)
            return a, b

        @plsc.parallel_loop(0, BE, 1, unroll=8)
        def edge(e):
            wbase = e & ~jnp.int32(15)
            lane = e & jnp.int32(15)
            cvec = cutv[pl.ds(wbase, 16)]
            scale = lane_perm(cvec, jnp.full((16,), 0, jnp.int32) + lane) * SCALE
            for h in range(H):
                dsl = pl.ds(h * HD, HD)
                kh, vh = unp(kvv[e, dsl])
                p = qv[e, dsl] * wv[e, dsl] * kh
                sv = scale * vh
                c_v[e, dsl] = jnp.sum(p) * sv
        pltpu.async_copy(c_v.at[pl.ds(0, BE)], acc.at[ridx], sem_sc, add=True)

    def drain_scatter():
        pltpu.make_async_copy(c_v.at[pl.ds(0, BE)], acc.at[pl.ds(0, BE)],
                              sem_sc).wait()

    issue_idx(0, sets[0])
    issue_idx(1, sets[1])
    issue_idx(2, sets[2])
    issue_w(0, wbufs[0])
    wait_idx(sets[0])
    issue_qkv(0, sets[0])
    wait_idx(sets[1])
    issue_qkv(1, sets[1])

    def six(t, carry):
        for u in range(6):
            i6 = 6 * t + u
            cur = sets[u % 3]
            wb = wbufs[u % 2]

            @pl.when(i6 + 2 < NBPW)
            def _():
                wait_idx(sets[(u + 2) % 3])
                issue_qkv(i6 + 2, sets[(u + 2) % 3])

            @pl.when(i6 + 1 < NBPW)
            def _():
                issue_w(i6 + 1, wbufs[(u + 1) % 2])

            @pl.when(i6 < NBPW)
            def _():
                wait_qkv(cur)
                wait_w(wb)
                @pl.when(i6 > 0)
                def _():
                    drain_scatter()
                compute_block(cur, wb)

            @pl.when(i6 + 3 < NBPW)
            def _():
                issue_idx(i6 + 3, cur)
        return carry
    lax.fori_loop(0, (NBPW + 5) // 6, six, 0)
    drain_scatter()

    plsc.subcore_barrier()

    def ochunk(j, carry):
        i = s + j * NS
        @pl.when(i < NZ)
        def _():
            r0 = i * ZC
            pltpu.sync_copy(acc.at[pl.ds(r0, ZC)], c_v)
            pltpu.sync_copy(c_v, out.at[c, pl.ds(r0, ZC)])
        return carry
    lax.fori_loop(0, (NZ + NS - 1) // NS, ochunk, 0)


def _sc_scatter(wp, qt, kvt, snd, rcv, cut):
    mesh = plsc.VectorSubcoreMesh(core_axis_name="c", subcore_axis_name="s")
    buf = lambda: [
        pltpu.VMEM((BE,), jnp.int32),
        pltpu.VMEM((BE,), jnp.int32),
        pltpu.VMEM((BE + 8,), jnp.float32),
        pltpu.VMEM((BE, D), jnp.float32),
        pltpu.VMEM((BE, D), jnp.uint32),
    ]
    f = functools.partial(
        pl.kernel,
        out_type=jax.ShapeDtypeStruct((NC, N, D), jnp.float32),
        mesh=mesh,
        compiler_params=pltpu.CompilerParams(needs_layout_passes=False),
        scratch_types=buf() + buf() + buf() + [
            pltpu.VMEM((BE, D), jnp.float32),
            pltpu.VMEM((BE, D), jnp.float32),
            pltpu.VMEM((ZC, D), jnp.float32),
            pltpu.VMEM_SHARED((N, D), jnp.float32),
            pltpu.SemaphoreType.DMA,
            pltpu.SemaphoreType.DMA,
            pltpu.SemaphoreType.DMA,
            pltpu.SemaphoreType.DMA,
            pltpu.SemaphoreType.DMA,
            pltpu.SemaphoreType.DMA,
            pltpu.SemaphoreType.DMA,
            pltpu.SemaphoreType.DMA,
            pltpu.SemaphoreType.DMA,
        ],
    )(_sc_body)
    return f(wp, qt, kvt, snd, rcv, cut)


def _sum_body(p, o):
    o[...] = p[0] + p[1]


def _sum_partials(parts):
    BN = 1000
    return pl.pallas_call(
        _sum_body,
        grid=(N // BN,),
        in_specs=[pl.BlockSpec((NC, BN, D), lambda i: (0, i, 0))],
        out_specs=pl.BlockSpec((BN, D), lambda i: (i, 0)),
        out_shape=jax.ShapeDtypeStruct((N, D), jnp.float32),
    )(parts)


def kernel(node_feats, edge_feats, chi_scalar, cutoffs, senders, receivers,
           W_rad1, b_rad1, W_rad2, b_rad2,
           W_sph1, b_sph1, W_sph2, b_sph2,
           Wq, Wk, Wv):
    wp = _edge_mlp(edge_feats, chi_scalar,
                   W_rad1, b_rad1.reshape(1, 64), W_rad2, b_rad2.reshape(1, D),
                   W_sph1, b_sph1.reshape(1, 64), W_sph2, b_sph2.reshape(1, D))
    qt, kvt = _qkv(node_feats, Wq, Wk, Wv)
    parts = _sc_scatter(wp, qt, kvt,
                        _to_i32(senders), _to_i32(receivers),
                        cutoffs.astype(jnp.float32))
    return _sum_partials(parts)


def _to_i32(x):
    if x.dtype == jnp.int64:
        return lax.bitcast_convert_type(x, jnp.int32)[..., 0]
    return x.astype(jnp.int32)
